# baseline (speedup 1.0000x reference)
import functools

import jax
import jax.numpy as jnp
from jax.experimental import pallas as pl
from jax.experimental.pallas import tpu as pltpu

N = 320000
D = 128
S = 10000

SB = 4000
NSTREAMS = 4


def _stats_kernel(*refs):
    x_refs, o_ref, acc_ref = refs[:NSTREAMS], refs[NSTREAMS], refs[NSTREAMS + 1]
    i = pl.program_id(0)

    @pl.when(i == 0)
    def _():
        acc_ref[...] = jnp.zeros_like(acc_ref)

    for r in x_refs:
        xb = r[...]
        acc_ref[0:1, :] += jnp.sum(xb, axis=0, keepdims=True)
        acc_ref[1:2, :] += jnp.sum(xb * xb, axis=0, keepdims=True)

    @pl.when(i == pl.num_programs(0) - 1)
    def _():
        o_ref[...] = acc_ref[...]


def _make_stats_spec(k):
    return pl.BlockSpec((SB, D), lambda i: (NSTREAMS * i + k, 0))


def _stats(x):
    return pl.pallas_call(
        _stats_kernel,
        grid=(N // (NSTREAMS * SB),),
        in_specs=[_make_stats_spec(k) for k in range(NSTREAMS)],
        out_specs=pl.BlockSpec((8, D), lambda i: (0, 0)),
        out_shape=jax.ShapeDtypeStruct((8, D), jnp.float32),
        scratch_shapes=[pltpu.VMEM((8, D), jnp.float32)],
    )(*([x] * NSTREAMS))


SUB = 3200
NSUB = 4
B = NSUB * SUB
W = 128
ACC_ROWS = 10496
NSTEPS = N // B
NB = N // SUB


def _pool_kernel(meta_ref, stats_ref, gamma_ref, beta_ref, unit_ref,
                 *refs):
    x_refs = refs[:NSUB]
    id_refs = refs[NSUB:2 * NSUB]
    o_ref = refs[2 * NSUB]
    acc_ref = refs[2 * NSUB + 1]
    rhs_refs = refs[2 * NSUB + 2:2 * NSUB + 2 + NSUB]
    i = pl.program_id(0)

    @pl.when(i == 0)
    def _():
        acc_ref[...] = jnp.zeros_like(acc_ref)
        for rr in rhs_refs:
            rr[:, D:] = jnp.ones((SUB, D), dtype=jnp.bfloat16)

    s = stats_ref[...]
    mean = s[0:1, :] / N
    var = s[1:2, :] / N - mean * mean
    rstd = jax.lax.rsqrt(var + 1e-5)
    scale = gamma_ref[...] * rstd
    shift = beta_ref[...] - mean * scale

    sub_iota = jax.lax.broadcasted_iota(jnp.int32, (W, SUB), 0)

    def do_sub(x_ref, ids_ref, rhs_ref, k):
        xb = x_ref[...]
        v = xb * scale + shift
        act = jnp.where(v > 0, v, jnp.exp(v) - 1.0).astype(jnp.bfloat16)
        rhs_ref[:, 0:D] = act
        rhs = rhs_ref[...]
        ids_row = ids_ref[0]

        anchor0 = (meta_ref[2 * (NSUB * i + k)] // 8) * 8
        span = meta_ref[2 * (NSUB * i + k) + 1]

        offs0 = jnp.broadcast_to(ids_row - anchor0, (W, SUB))
        onehot = jnp.where(
            sub_iota == offs0, 1.0, 0.0).astype(jnp.bfloat16)
        contrib = jax.lax.dot_general(
            onehot, rhs, (((1,), (0,)), ((), ())),
            preferred_element_type=jnp.float32)
        acc_ref[pl.ds(anchor0, W), :] += contrib

        @pl.when(span >= W)
        def _():
            def window_pass(rem_i, anchor):
                offs = ids_row - anchor
                sel = (rem_i > 0) & (offs >= 0) & (offs < W)
                offs_m = jnp.broadcast_to(
                    jnp.where(sel, offs, jnp.int32(-1)), (W, SUB))
                oh = jnp.where(
                    sub_iota == offs_m, 1.0, 0.0).astype(jnp.bfloat16)
                c = jax.lax.dot_general(
                    oh, rhs, (((1,), (0,)), ((), ())),
                    preferred_element_type=jnp.float32)
                acc_ref[pl.ds(anchor, W), :] += c
                return jnp.where(sel, jnp.int32(0), rem_i)

            def cond(carry):
                rem_i, _ = carry
                return jnp.max(rem_i) > 0

            def body(carry):
                rem_i, _ = carry
                masked = jnp.where(rem_i > 0, ids_row, jnp.int32(1 << 30))
                anchor = (jnp.min(masked) // 8) * 8
                return window_pass(rem_i, anchor), anchor

            rem0 = jnp.where(ids_row - anchor0 >= W, 1, 0).astype(jnp.int32)
            jax.lax.while_loop(cond, body, (rem0, jnp.int32(0)))

    for k in range(NSUB):
        do_sub(x_refs[k], id_refs[k], rhs_refs[k], k)

    @pl.when(i == pl.num_programs(0) - 1)
    def _():
        sums = acc_ref[0:S, 0:D]
        counts = acc_ref[0:S, D:D + 1]
        o_ref[...] = sums * unit_ref[0, 0] / jnp.maximum(counts, 1.0)


def _make_pool_x_spec(k):
    return pl.BlockSpec((SUB, D), lambda i, a: (NSUB * i + k, 0))


def _make_pool_id_spec(k):
    return pl.BlockSpec((1, 1, SUB), lambda i, a: (NSUB * i + k, 0, 0))


def _pool(meta, stats, gamma, beta, unit, x, ids3):
    grid_spec = pltpu.PrefetchScalarGridSpec(
        num_scalar_prefetch=1,
        grid=(NSTEPS,),
        in_specs=[
            pl.BlockSpec((8, D), lambda i, a: (0, 0)),
            pl.BlockSpec((1, D), lambda i, a: (0, 0)),
            pl.BlockSpec((1, D), lambda i, a: (0, 0)),
            pl.BlockSpec((1, 1), lambda i, a: (0, 0)),
        ] + [_make_pool_x_spec(k) for k in range(NSUB)]
          + [_make_pool_id_spec(k) for k in range(NSUB)],
        out_specs=pl.BlockSpec((S, D), lambda i, a: (0, 0)),
        scratch_shapes=[pltpu.VMEM((ACC_ROWS, 2 * D), jnp.float32)]
        + [pltpu.VMEM((SUB, 2 * D), jnp.bfloat16) for _ in range(NSUB)],
    )
    return pl.pallas_call(
        _pool_kernel,
        grid_spec=grid_spec,
        out_shape=jax.ShapeDtypeStruct((S, D), jnp.float32),
    )(meta, stats, gamma, beta, unit,
      *([x] * NSUB), *([ids3] * NSUB))


@functools.partial(jax.jit, static_argnames=())
def kernel(x, ids, num_seg, gamma, beta):
    stats = _stats(x)
    ids32 = ids.astype(jnp.int32)
    anchors = (ids32[::SUB] // 8) * 8
    spans = ids32[SUB - 1::SUB] - anchors
    meta = jnp.stack([anchors, spans], axis=1).reshape(-1)
    unit = (jnp.asarray(num_seg, dtype=jnp.float32) / S).reshape(1, 1)
    ids3 = ids32.reshape(NB, 1, SUB)
    return _pool(meta, stats, gamma.reshape(1, D), beta.reshape(1, D),
                 unit, x, ids3)


# speedup vs baseline: 1.1423x; 1.1423x over previous
import functools

import jax
import jax.numpy as jnp
from jax.experimental import pallas as pl
from jax.experimental.pallas import tpu as pltpu

N = 320000
D = 128
S = 10000

SB = 4000
NSTREAMS = 4


def _stats_kernel(*refs):
    x_refs, o_ref, acc_ref = refs[:NSTREAMS], refs[NSTREAMS], refs[NSTREAMS + 1]
    i = pl.program_id(0)

    @pl.when(i == 0)
    def _():
        acc_ref[...] = jnp.zeros_like(acc_ref)

    for r in x_refs:
        xb = r[...]
        acc_ref[0:1, :] += jnp.sum(xb, axis=0, keepdims=True)
        acc_ref[1:2, :] += jnp.sum(xb * xb, axis=0, keepdims=True)

    @pl.when(i == pl.num_programs(0) - 1)
    def _():
        o_ref[...] = acc_ref[...]


def _make_stats_spec(k):
    return pl.BlockSpec((SB, D), lambda i: (NSTREAMS * i + k, 0))


def _stats(x):
    return pl.pallas_call(
        _stats_kernel,
        grid=(N // (NSTREAMS * SB),),
        in_specs=[_make_stats_spec(k) for k in range(NSTREAMS)],
        out_specs=pl.BlockSpec((8, D), lambda i: (0, 0)),
        out_shape=jax.ShapeDtypeStruct((8, D), jnp.float32),
        scratch_shapes=[pltpu.VMEM((8, D), jnp.float32)],
    )(*([x] * NSTREAMS))


SUB = 3200
NSUB = 4
B = NSUB * SUB
W = 128
ACC_ROWS = 10496
NSTEPS = N // B
NB = N // SUB


def _pool_kernel(meta_ref, stats_ref, gamma_ref, beta_ref, unit_ref,
                 *refs):
    x_refs = refs[:NSUB]
    id_refs = refs[NSUB:2 * NSUB]
    o_ref = refs[2 * NSUB]
    acc_ref = refs[2 * NSUB + 1]
    i = pl.program_id(0)

    @pl.when(i == 0)
    def _():
        acc_ref[...] = jnp.zeros_like(acc_ref)

    s = stats_ref[...]
    mean = s[0:1, :] / N
    var = s[1:2, :] / N - mean * mean
    rstd = jax.lax.rsqrt(var + 1e-5)
    scale = gamma_ref[...] * rstd
    shift = beta_ref[...] - mean * scale

    sub_iota = jax.lax.broadcasted_iota(jnp.int32, (W, SUB), 0)

    def do_sub(x_ref, ids_ref, k):
        xb = x_ref[...]
        v = xb * scale + shift
        act = jnp.where(v > 0, v, jnp.exp(v) - 1.0).astype(jnp.bfloat16)
        rhs = jnp.concatenate(
            [act, jnp.ones((SUB, D), dtype=jnp.bfloat16)], axis=1)
        ids_row = ids_ref[0]

        anchor0 = (meta_ref[2 * (NSUB * i + k)] // 8) * 8
        span = meta_ref[2 * (NSUB * i + k) + 1]

        offs0 = jnp.broadcast_to(ids_row - anchor0, (W, SUB))
        onehot = jnp.where(
            sub_iota == offs0, 1.0, 0.0).astype(jnp.bfloat16)
        contrib = jax.lax.dot_general(
            onehot, rhs, (((1,), (0,)), ((), ())),
            preferred_element_type=jnp.float32)
        acc_ref[pl.ds(anchor0, W), :] += contrib

        @pl.when(span >= W)
        def _():
            def window_pass(rem_i, anchor):
                offs = ids_row - anchor
                sel = (rem_i > 0) & (offs >= 0) & (offs < W)
                offs_m = jnp.broadcast_to(
                    jnp.where(sel, offs, jnp.int32(-1)), (W, SUB))
                oh = jnp.where(
                    sub_iota == offs_m, 1.0, 0.0).astype(jnp.bfloat16)
                c = jax.lax.dot_general(
                    oh, rhs, (((1,), (0,)), ((), ())),
                    preferred_element_type=jnp.float32)
                acc_ref[pl.ds(anchor, W), :] += c
                return jnp.where(sel, jnp.int32(0), rem_i)

            def cond(carry):
                rem_i, _ = carry
                return jnp.max(rem_i) > 0

            def body(carry):
                rem_i, _ = carry
                masked = jnp.where(rem_i > 0, ids_row, jnp.int32(1 << 30))
                anchor = (jnp.min(masked) // 8) * 8
                return window_pass(rem_i, anchor), anchor

            rem0 = jnp.where(ids_row - anchor0 >= W, 1, 0).astype(jnp.int32)
            jax.lax.while_loop(cond, body, (rem0, jnp.int32(0)))

    for k in range(NSUB):
        do_sub(x_refs[k], id_refs[k], k)

    @pl.when(i == pl.num_programs(0) - 1)
    def _():
        sums = acc_ref[0:S, 0:D]
        counts = acc_ref[0:S, D:D + 1]
        o_ref[...] = sums * unit_ref[0, 0] / jnp.maximum(counts, 1.0)


def _make_pool_x_spec(k):
    return pl.BlockSpec((SUB, D), lambda i, a: (NSUB * i + k, 0))


def _make_pool_id_spec(k):
    return pl.BlockSpec((1, 1, SUB), lambda i, a: (NSUB * i + k, 0, 0))


def _pool(meta, stats, gamma, beta, unit, x, ids3):
    grid_spec = pltpu.PrefetchScalarGridSpec(
        num_scalar_prefetch=1,
        grid=(NSTEPS,),
        in_specs=[
            pl.BlockSpec((8, D), lambda i, a: (0, 0)),
            pl.BlockSpec((1, D), lambda i, a: (0, 0)),
            pl.BlockSpec((1, D), lambda i, a: (0, 0)),
            pl.BlockSpec((1, 1), lambda i, a: (0, 0)),
        ] + [_make_pool_x_spec(k) for k in range(NSUB)]
          + [_make_pool_id_spec(k) for k in range(NSUB)],
        out_specs=pl.BlockSpec((S, D), lambda i, a: (0, 0)),
        scratch_shapes=[pltpu.VMEM((ACC_ROWS, 2 * D), jnp.float32)],
    )
    return pl.pallas_call(
        _pool_kernel,
        grid_spec=grid_spec,
        out_shape=jax.ShapeDtypeStruct((S, D), jnp.float32),
    )(meta, stats, gamma, beta, unit,
      *([x] * NSUB), *([ids3] * NSUB))


@functools.partial(jax.jit, static_argnames=())
def kernel(x, ids, num_seg, gamma, beta):
    stats = _stats(x)
    ids32 = ids.astype(jnp.int32)
    anchors = (ids32[::SUB] // 8) * 8
    spans = ids32[SUB - 1::SUB] - anchors
    meta = jnp.stack([anchors, spans], axis=1).reshape(-1)
    unit = (jnp.asarray(num_seg, dtype=jnp.float32) / S).reshape(1, 1)
    ids3 = ids32.reshape(NB, 1, SUB)
    return _pool(meta, stats, gamma.reshape(1, D), beta.reshape(1, D),
                 unit, x, ids3)
